# Initial kernel scaffold; baseline (speedup 1.0000x reference)
import jax
import jax.numpy as jnp
from jax.experimental import pallas as pl


def kernel(x, edge_index, W, B):
    raise NotImplementedError("write your pallas kernel here")


# speedup vs baseline: 41.4446x; 41.4446x over previous
import functools

import jax
import jax.numpy as jnp
from jax import lax
from jax.experimental import pallas as pl
from jax.experimental.pallas import tpu as pltpu
from jax.experimental.pallas import tpu_sc as plsc

N_NODES = 100000
N_EDGES = 6400000
D_IN = 10
DP = 16

NC = 2
NS = 16
NW = NC * NS

ROWS_PER_STREAM = 128
STREAMS_PER_CHUNK = 10
CHUNK = ROWS_PER_STREAM * STREAMS_PER_CHUNK
NCHUNKS = N_EDGES // CHUNK
KMAX = -(-NCHUNKS // NW)

ZCHUNK = 1280
NZROUNDS = -(-N_NODES // (NS * ZCHUNK))
N_FULL_Z = N_NODES // ZCHUNK
Z_TAIL = N_NODES - N_FULL_Z * ZCHUNK

_MESH = plsc.VectorSubcoreMesh(core_axis_name="c", subcore_axis_name="s")


@functools.partial(
    pl.kernel,
    out_type=jax.ShapeDtypeStruct((NC * N_NODES, DP), jnp.float32),
    mesh=_MESH,
    scratch_types=[
        pltpu.VMEM((STREAMS_PER_CHUNK, ROWS_PER_STREAM), jnp.int32),
        pltpu.VMEM((STREAMS_PER_CHUNK, ROWS_PER_STREAM), jnp.int32),
        pltpu.VMEM((CHUNK, DP), jnp.float32),
        pltpu.VMEM_SHARED((N_NODES, DP), jnp.float32),
        pltpu.SemaphoreType.DMA,
    ],
    compiler_params=pltpu.CompilerParams(use_tc_tiling_on_sc=False),
)
def _edge_aggregate(xpad_hbm, src_hbm, dst_hbm, zeros_hbm, out_hbm,
                    srcv, dstv, rows, acc, gsem):
    c = lax.axis_index("c")
    s = lax.axis_index("s")
    wid = c * NS + s

    @pl.loop(0, NZROUNDS)
    def _zero(k):
        m = k * NS + s
        off = m * ZCHUNK

        @pl.when(off + ZCHUNK <= N_NODES)
        def _():
            pltpu.sync_copy(zeros_hbm, acc.at[pl.ds(off, ZCHUNK)])

        @pl.when(m == N_FULL_Z)
        def _():
            pltpu.sync_copy(zeros_hbm.at[pl.ds(0, Z_TAIL)],
                            acc.at[pl.ds(N_FULL_Z * ZCHUNK, Z_TAIL)])

    plsc.subcore_barrier()

    @pl.loop(0, KMAX)
    def _chunk(k):
        cid = k * NW + wid

        @pl.when(cid < NCHUNKS)
        def _():
            pltpu.sync_copy(src_hbm.at[cid], srcv)
            pltpu.sync_copy(dst_hbm.at[cid], dstv)
            copies = []
            for j in range(STREAMS_PER_CHUNK):
                copies.append(pltpu.async_copy(
                    xpad_hbm.at[srcv.at[j]],
                    rows.at[pl.ds(j * ROWS_PER_STREAM, ROWS_PER_STREAM)],
                    gsem))
            for cp in copies:
                cp.wait()
            for j in range(STREAMS_PER_CHUNK):
                pltpu.sync_copy(
                    rows.at[pl.ds(j * ROWS_PER_STREAM, ROWS_PER_STREAM)],
                    acc.at[dstv.at[j]], add=True)

    plsc.subcore_barrier()

    @pl.loop(0, NZROUNDS)
    def _drain(k):
        m = k * NS + s
        off = m * ZCHUNK

        @pl.when(off + ZCHUNK <= N_NODES)
        def _():
            pltpu.sync_copy(acc.at[pl.ds(off, ZCHUNK)],
                            out_hbm.at[pl.ds(c * N_NODES + off, ZCHUNK)])

        @pl.when(m == N_FULL_Z)
        def _():
            pltpu.sync_copy(
                acc.at[pl.ds(N_FULL_Z * ZCHUNK, Z_TAIL)],
                out_hbm.at[pl.ds(c * N_NODES + N_FULL_Z * ZCHUNK, Z_TAIL)])


BR = 2000
GRID = N_NODES // BR


def _finish_body(p_ref, w_ref, b_ref, o_ref):
    sums = p_ref[0] + p_ref[1]
    cnt = sums[:, D_IN:D_IN + 1]
    agg = sums / cnt
    m = w_ref[...] + b_ref[...]
    out = lax.dot_general(agg, m, (((1,), (1,)), ((), ())),
                          preferred_element_type=jnp.float32)
    o_ref[...] = jnp.maximum(out, 0.0)


def kernel(x, edge_index, W, B):
    xpad = jnp.zeros((N_NODES, DP), jnp.float32)
    xpad = xpad.at[:, :D_IN].set(x).at[:, D_IN].set(1.0)
    src3 = edge_index[0].reshape(NCHUNKS, STREAMS_PER_CHUNK, ROWS_PER_STREAM)
    dst3 = edge_index[1].reshape(NCHUNKS, STREAMS_PER_CHUNK, ROWS_PER_STREAM)
    zeros2 = jnp.zeros((ZCHUNK, DP), jnp.float32)

    partials = _edge_aggregate(xpad, src3, dst3, zeros2)
    partials3 = partials.reshape(NC, N_NODES, DP)

    wp = jnp.zeros((DP, DP), jnp.float32).at[:D_IN, :D_IN].set(W)
    bp = jnp.zeros((DP, DP), jnp.float32).at[:D_IN, :D_IN].set(B)

    out16 = pl.pallas_call(
        _finish_body,
        grid=(GRID,),
        in_specs=[
            pl.BlockSpec((NC, BR, DP), lambda i: (0, i, 0)),
            pl.BlockSpec((DP, DP), lambda i: (0, 0)),
            pl.BlockSpec((DP, DP), lambda i: (0, 0)),
        ],
        out_specs=pl.BlockSpec((BR, DP), lambda i: (i, 0)),
        out_shape=jax.ShapeDtypeStruct((N_NODES, DP), jnp.float32),
    )(partials3, wp, bp)
    return out16[:, :D_IN]


# speedup vs baseline: 54.3590x; 1.3116x over previous
import functools

import jax
import jax.numpy as jnp
from jax import lax
from jax.experimental import pallas as pl
from jax.experimental.pallas import tpu as pltpu
from jax.experimental.pallas import tpu_sc as plsc

N_NODES = 100000
N_EDGES = 6400000
D_IN = 10
DP = 16

NC = 2
NS = 16
NW = NC * NS

S = 128
J = 6
CHUNK = J * S
NCH = 262
E_PAD = NW * NCH * CHUNK
N_DUMMY = 1024
NPAD = N_NODES + N_DUMMY

ZCHUNK = 1280
NZROUNDS = -(-N_NODES // (NS * ZCHUNK))
N_FULL_Z = N_NODES // ZCHUNK
Z_TAIL = N_NODES - N_FULL_Z * ZCHUNK

_MESH = plsc.VectorSubcoreMesh(core_axis_name="c", subcore_axis_name="s")


@functools.partial(
    pl.kernel,
    out_type=jax.ShapeDtypeStruct((NC * N_NODES, DP), jnp.float32),
    mesh=_MESH,
    scratch_types=[
        pltpu.VMEM((J, S), jnp.int32),
        pltpu.VMEM((J, S), jnp.int32),
        pltpu.VMEM((J, S), jnp.int32),
        pltpu.VMEM((J, S), jnp.int32),
        pltpu.VMEM((CHUNK, DP), jnp.float32),
        pltpu.VMEM((CHUNK, DP), jnp.float32),
        pltpu.VMEM_SHARED((NPAD, DP), jnp.float32),
        pltpu.SemaphoreType.DMA,
        pltpu.SemaphoreType.DMA,
        pltpu.SemaphoreType.DMA,
    ],
    compiler_params=pltpu.CompilerParams(use_tc_tiling_on_sc=False),
)
def _edge_aggregate(xpad_hbm, src_hbm, dst_hbm, zeros_hbm, out_hbm,
                    srcv0, srcv1, dstv0, dstv1, rows0, rows1,
                    acc, gsem, ssem, isem):
    c = lax.axis_index("c")
    s = lax.axis_index("s")
    wid = c * NS + s
    base = wid * NCH

    srcv = (srcv0, srcv1)
    dstv = (dstv0, dstv1)
    rows = (rows0, rows1)

    @pl.loop(0, NZROUNDS)
    def _zero(k):
        m = k * NS + s
        off = m * ZCHUNK

        @pl.when(off + ZCHUNK <= N_NODES)
        def _():
            pltpu.sync_copy(zeros_hbm, acc.at[pl.ds(off, ZCHUNK)])

        @pl.when(m == N_FULL_Z)
        def _():
            pltpu.sync_copy(zeros_hbm.at[pl.ds(0, Z_TAIL)],
                            acc.at[pl.ds(N_FULL_Z * ZCHUNK, Z_TAIL)])

    plsc.subcore_barrier()

    def fire_gathers(p):
        for j in range(J):
            pltpu.async_copy(xpad_hbm.at[srcv[p].at[j]],
                             rows[p].at[pl.ds(j * S, S)], gsem)

    def drain_gathers(p):
        for j in range(J):
            pltpu.make_async_copy(xpad_hbm.at[srcv[p].at[j]],
                                  rows[p].at[pl.ds(j * S, S)], gsem).wait()

    def fire_scatters(p):
        for j in range(J):
            pltpu.async_copy(rows[p].at[pl.ds(j * S, S)],
                             acc.at[dstv[p].at[j]], ssem, add=True)

    def drain_scatters(p):
        for j in range(J):
            pltpu.make_async_copy(rows[p].at[pl.ds(j * S, S)],
                                  acc.at[dstv[p].at[j]], ssem).wait()

    def fire_idx(cid, p):
        pltpu.async_copy(src_hbm.at[cid], srcv[p], isem)
        pltpu.async_copy(dst_hbm.at[cid], dstv[p], isem)

    def drain_idx(cid, p):
        pltpu.make_async_copy(src_hbm.at[cid], srcv[p], isem).wait()
        pltpu.make_async_copy(dst_hbm.at[cid], dstv[p], isem).wait()

    pltpu.sync_copy(src_hbm.at[base], srcv[0])
    pltpu.sync_copy(dst_hbm.at[base], dstv[0])
    fire_gathers(0)

    @pl.loop(0, NCH // 2)
    def _pair(t):
        a = base + 2 * t

        @pl.when(t >= 1)
        def _():
            drain_scatters(1)
        fire_idx(a + 1, 1)
        drain_gathers(0)
        fire_scatters(0)
        drain_idx(a + 1, 1)
        fire_gathers(1)

        drain_scatters(0)

        @pl.when(t < NCH // 2 - 1)
        def _():
            fire_idx(a + 2, 0)
        drain_gathers(1)
        fire_scatters(1)

        @pl.when(t < NCH // 2 - 1)
        def _():
            drain_idx(a + 2, 0)
            fire_gathers(0)

    drain_scatters(1)

    plsc.subcore_barrier()

    @pl.loop(0, NZROUNDS)
    def _drain(k):
        m = k * NS + s
        off = m * ZCHUNK

        @pl.when(off + ZCHUNK <= N_NODES)
        def _():
            pltpu.sync_copy(acc.at[pl.ds(off, ZCHUNK)],
                            out_hbm.at[pl.ds(c * N_NODES + off, ZCHUNK)])

        @pl.when(m == N_FULL_Z)
        def _():
            pltpu.sync_copy(
                acc.at[pl.ds(N_FULL_Z * ZCHUNK, Z_TAIL)],
                out_hbm.at[pl.ds(c * N_NODES + N_FULL_Z * ZCHUNK, Z_TAIL)])


BR = 2000
GRID = N_NODES // BR


def _finish_body(p_ref, w_ref, b_ref, o_ref):
    sums = p_ref[0] + p_ref[1]
    cnt = sums[:, D_IN:D_IN + 1]
    agg = sums / cnt
    m = w_ref[...] + b_ref[...]
    out = lax.dot_general(agg, m, (((1,), (1,)), ((), ())),
                          preferred_element_type=jnp.float32)
    o_ref[...] = jnp.maximum(out, 0.0)


def kernel(x, edge_index, W, B):
    xpad = jnp.zeros((NPAD, DP), jnp.float32)
    xpad = xpad.at[:N_NODES, :D_IN].set(x).at[:N_NODES, D_IN].set(1.0)

    pad_e = E_PAD - N_EDGES
    pad_idx = N_NODES + (jnp.arange(pad_e, dtype=jnp.int32) % N_DUMMY)
    src = jnp.concatenate([edge_index[0], pad_idx])
    dst = jnp.concatenate([edge_index[1], pad_idx])
    src3 = src.reshape(NW * NCH, J, S)
    dst3 = dst.reshape(NW * NCH, J, S)
    zeros2 = jnp.zeros((ZCHUNK, DP), jnp.float32)

    partials = _edge_aggregate(xpad, src3, dst3, zeros2)
    partials3 = partials.reshape(NC, N_NODES, DP)

    wp = jnp.zeros((DP, DP), jnp.float32).at[:D_IN, :D_IN].set(W)
    bp = jnp.zeros((DP, DP), jnp.float32).at[:D_IN, :D_IN].set(B)

    out16 = pl.pallas_call(
        _finish_body,
        grid=(GRID,),
        in_specs=[
            pl.BlockSpec((NC, BR, DP), lambda i: (0, i, 0)),
            pl.BlockSpec((DP, DP), lambda i: (0, 0)),
            pl.BlockSpec((DP, DP), lambda i: (0, 0)),
        ],
        out_specs=pl.BlockSpec((BR, DP), lambda i: (i, 0)),
        out_shape=jax.ShapeDtypeStruct((N_NODES, DP), jnp.float32),
    )(partials3, wp, bp)
    return out16[:, :D_IN]


# speedup vs baseline: 75.3276x; 1.3857x over previous
import functools

import jax
import jax.numpy as jnp
import numpy as np
from jax import lax
from jax.experimental import pallas as pl
from jax.experimental.pallas import tpu as pltpu
from jax.experimental.pallas import tpu_sc as plsc

N_NODES = 100000
N_EDGES = 6400000
D_IN = 10
DP = 16

NC = 2
NS = 16
NW = NC * NS

S = 128
J = 6
CHUNK = J * S
NCH = 262
E_PAD = NW * NCH * CHUNK
EP_ROWS = E_PAD // S
N_DUMMY = 1024
NPAD = N_NODES + N_DUMMY
XROWS = NPAD * DP // 128

ZCHUNK = 1280
NZROUNDS = -(-N_NODES // (NS * ZCHUNK))
N_FULL_Z = N_NODES // ZCHUNK
Z_TAIL = N_NODES - N_FULL_Z * ZCHUNK

_MESH = plsc.VectorSubcoreMesh(core_axis_name="c", subcore_axis_name="s")

_PSEL = np.zeros((128, 128), np.float32)
for _i in range(128):
    _PSEL[16 * (_i // 16) + D_IN, _i] = 1.0


EB_ROWS = 384
EB = EB_ROWS * S
EGRID = EP_ROWS // EB_ROWS


def _edge_prep_body(ei_ref, s_ref, d_ref):
    g = pl.program_id(0)
    eid = (g * EB
           + lax.broadcasted_iota(jnp.int32, (EB_ROWS, S), 0) * S
           + lax.broadcasted_iota(jnp.int32, (EB_ROWS, S), 1))
    padv = N_NODES + (eid & (N_DUMMY - 1))
    real = eid < N_EDGES
    s_ref[...] = jnp.where(real, ei_ref[0].reshape(EB_ROWS, S), padv)
    d_ref[...] = jnp.where(real, ei_ref[1].reshape(EB_ROWS, S), padv)


@functools.partial(
    pl.kernel,
    out_type=jax.ShapeDtypeStruct((NC * N_NODES, DP), jnp.float32),
    mesh=_MESH,
    scratch_types=[
        pltpu.VMEM((J, S), jnp.int32),
        pltpu.VMEM((J, S), jnp.int32),
        pltpu.VMEM((J, S), jnp.int32),
        pltpu.VMEM((J, S), jnp.int32),
        pltpu.VMEM((CHUNK, DP), jnp.float32),
        pltpu.VMEM((CHUNK, DP), jnp.float32),
        pltpu.VMEM_SHARED((NPAD, DP), jnp.float32),
        pltpu.SemaphoreType.DMA,
        pltpu.SemaphoreType.DMA,
        pltpu.SemaphoreType.DMA,
    ],
    compiler_params=pltpu.CompilerParams(use_tc_tiling_on_sc=False),
)
def _edge_aggregate(xpad_hbm, src_hbm, dst_hbm, zeros_hbm, out_hbm,
                    srcv0, srcv1, dstv0, dstv1, rows0, rows1,
                    acc, gsem, ssem, isem):
    c = lax.axis_index("c")
    s = lax.axis_index("s")
    wid = c * NS + s
    base = wid * NCH

    srcv = (srcv0, srcv1)
    dstv = (dstv0, dstv1)
    rows = (rows0, rows1)

    @pl.loop(0, NZROUNDS)
    def _zero(k):
        m = k * NS + s
        off = m * ZCHUNK

        @pl.when(off + ZCHUNK <= N_NODES)
        def _():
            pltpu.sync_copy(zeros_hbm, acc.at[pl.ds(off, ZCHUNK)])

        @pl.when(m == N_FULL_Z)
        def _():
            pltpu.sync_copy(zeros_hbm.at[pl.ds(0, Z_TAIL)],
                            acc.at[pl.ds(N_FULL_Z * ZCHUNK, Z_TAIL)])

    plsc.subcore_barrier()

    def fire_gathers(p):
        for j in range(J):
            pltpu.async_copy(xpad_hbm.at[srcv[p].at[j]],
                             rows[p].at[pl.ds(j * S, S)], gsem)

    def drain_gathers(p):
        for j in range(J):
            pltpu.make_async_copy(xpad_hbm.at[srcv[p].at[j]],
                                  rows[p].at[pl.ds(j * S, S)], gsem).wait()

    def fire_scatters(p):
        for j in range(J):
            pltpu.async_copy(rows[p].at[pl.ds(j * S, S)],
                             acc.at[dstv[p].at[j]], ssem, add=True)

    def drain_scatters(p):
        for j in range(J):
            pltpu.make_async_copy(rows[p].at[pl.ds(j * S, S)],
                                  acc.at[dstv[p].at[j]], ssem).wait()

    def fire_idx(cid, p):
        pltpu.async_copy(src_hbm.at[pl.ds(cid * J, J)], srcv[p], isem)
        pltpu.async_copy(dst_hbm.at[pl.ds(cid * J, J)], dstv[p], isem)

    def drain_idx(cid, p):
        pltpu.make_async_copy(src_hbm.at[pl.ds(cid * J, J)], srcv[p],
                              isem).wait()
        pltpu.make_async_copy(dst_hbm.at[pl.ds(cid * J, J)], dstv[p],
                              isem).wait()

    pltpu.sync_copy(src_hbm.at[pl.ds(base * J, J)], srcv[0])
    pltpu.sync_copy(dst_hbm.at[pl.ds(base * J, J)], dstv[0])
    fire_gathers(0)

    @pl.loop(0, NCH // 2)
    def _pair(t):
        a = base + 2 * t

        @pl.when(t >= 1)
        def _():
            drain_scatters(1)
        fire_idx(a + 1, 1)
        drain_gathers(0)
        fire_scatters(0)
        drain_idx(a + 1, 1)
        fire_gathers(1)

        drain_scatters(0)

        @pl.when(t < NCH // 2 - 1)
        def _():
            fire_idx(a + 2, 0)
        drain_gathers(1)
        fire_scatters(1)

        @pl.when(t < NCH // 2 - 1)
        def _():
            drain_idx(a + 2, 0)
            fire_gathers(0)

    drain_scatters(1)

    plsc.subcore_barrier()

    @pl.loop(0, NZROUNDS)
    def _drain(k):
        m = k * NS + s
        off = m * ZCHUNK

        @pl.when(off + ZCHUNK <= N_NODES)
        def _():
            pltpu.sync_copy(acc.at[pl.ds(off, ZCHUNK)],
                            out_hbm.at[pl.ds(c * N_NODES + off, ZCHUNK)])

        @pl.when(m == N_FULL_Z)
        def _():
            pltpu.sync_copy(
                acc.at[pl.ds(N_FULL_Z * ZCHUNK, Z_TAIL)],
                out_hbm.at[pl.ds(c * N_NODES + N_FULL_Z * ZCHUNK, Z_TAIL)])


PROWS = N_NODES * DP // 128
FB_ROWS = 512
FGRID = -(-PROWS // FB_ROWS)


def _finish_body(p_ref, psel_ref, mbd_ref, o_ref):
    sums = p_ref[0] + p_ref[1]
    cnt = jnp.dot(sums, psel_ref[...],
                  preferred_element_type=jnp.float32)
    agg = sums / cnt
    out = jnp.dot(agg, mbd_ref[...], preferred_element_type=jnp.float32)
    o_ref[...] = jnp.maximum(out, 0.0)


def kernel(x, edge_index, W, B):
    se, de = pl.pallas_call(
        _edge_prep_body,
        grid=(EGRID,),
        in_specs=[pl.BlockSpec((2, EB), lambda g: (0, g))],
        out_specs=[pl.BlockSpec((EB_ROWS, S), lambda g: (g, 0)),
                   pl.BlockSpec((EB_ROWS, S), lambda g: (g, 0))],
        out_shape=[jax.ShapeDtypeStruct((EP_ROWS, S), jnp.int32),
                   jax.ShapeDtypeStruct((EP_ROWS, S), jnp.int32)],
    )(edge_index)

    x16 = jnp.concatenate(
        [x, jnp.ones((N_NODES, 1), jnp.float32),
         jnp.zeros((N_NODES, DP - D_IN - 1), jnp.float32)], axis=1)
    xpad = jnp.pad(x16, ((0, N_DUMMY), (0, 0)))

    zeros2 = jnp.zeros((ZCHUNK, DP), jnp.float32)
    partials = _edge_aggregate(xpad, se, de, zeros2)
    p128 = partials.reshape(2, PROWS, 128)

    mp = jnp.zeros((DP, DP), jnp.float32)
    mp = mp.at[:D_IN, :D_IN].set((W + B).T)
    mbd = jnp.kron(jnp.eye(8, dtype=jnp.float32), mp)
    psel = jnp.asarray(_PSEL)

    out128 = pl.pallas_call(
        _finish_body,
        grid=(FGRID,),
        in_specs=[
            pl.BlockSpec((2, FB_ROWS, 128), lambda g: (0, g, 0)),
            pl.BlockSpec((128, 128), lambda g: (0, 0)),
            pl.BlockSpec((128, 128), lambda g: (0, 0)),
        ],
        out_specs=pl.BlockSpec((FB_ROWS, 128), lambda g: (g, 0)),
        out_shape=jax.ShapeDtypeStruct((PROWS, 128), jnp.float32),
    )(p128, psel, mbd)
    return out128.reshape(N_NODES, DP)[:, :D_IN]


# speedup vs baseline: 81.2147x; 1.0782x over previous
import functools

import jax
import jax.numpy as jnp
import numpy as np
from jax import lax
from jax.experimental import pallas as pl
from jax.experimental.pallas import tpu as pltpu
from jax.experimental.pallas import tpu_sc as plsc

N_NODES = 100000
N_EDGES = 6400000
D_IN = 10
DP = 16

NC = 2
NS = 16
NW = NC * NS

S = 128
J = 5
CHUNK = J * S
NCHUNKS = N_EDGES // CHUNK
KFULL = 312
NREM = NCHUNKS - KFULL * NW

ZCHUNK = 1280
NZROUNDS = -(-N_NODES // (NS * ZCHUNK))
N_FULL_Z = N_NODES // ZCHUNK
Z_TAIL = N_NODES - N_FULL_Z * ZCHUNK

_MESH = plsc.VectorSubcoreMesh(core_axis_name="c", subcore_axis_name="s")

_PSEL = np.zeros((128, 128), np.float32)
for _i in range(128):
    _PSEL[16 * (_i // 16) + D_IN, _i] = 1.0


@functools.partial(
    pl.kernel,
    out_type=jax.ShapeDtypeStruct((NC * N_NODES, DP), jnp.float32),
    mesh=_MESH,
    scratch_types=[
        pltpu.VMEM((CHUNK,), jnp.int32),
        pltpu.VMEM((CHUNK,), jnp.int32),
        pltpu.VMEM((J, S), jnp.int32),
        pltpu.VMEM((J, S), jnp.int32),
        pltpu.VMEM((CHUNK, DP), jnp.float32),
        pltpu.VMEM((CHUNK, DP), jnp.float32),
        pltpu.VMEM_SHARED((N_NODES, DP), jnp.float32),
        pltpu.SemaphoreType.DMA,
        pltpu.SemaphoreType.DMA,
        pltpu.SemaphoreType.DMA,
    ],
    compiler_params=pltpu.CompilerParams(use_tc_tiling_on_sc=False),
)
def _edge_aggregate(xpad_hbm, ei_hbm, zeros_hbm, out_hbm,
                    srcv0, srcv1, dstv0, dstv1, rows0, rows1,
                    acc, gsem, ssem, isem):
    c = lax.axis_index("c")
    s = lax.axis_index("s")
    wid = c * NS + s

    srcv = (srcv0, srcv1)
    dstv = (dstv0, dstv1)
    rows = (rows0, rows1)

    @pl.loop(0, NZROUNDS)
    def _zero(k):
        m = k * NS + s
        off = m * ZCHUNK

        @pl.when(off + ZCHUNK <= N_NODES)
        def _():
            pltpu.sync_copy(zeros_hbm, acc.at[pl.ds(off, ZCHUNK)])

        @pl.when(m == N_FULL_Z)
        def _():
            pltpu.sync_copy(zeros_hbm.at[pl.ds(0, Z_TAIL)],
                            acc.at[pl.ds(N_FULL_Z * ZCHUNK, Z_TAIL)])

    plsc.subcore_barrier()

    def fire_gathers(p):
        for j in range(J):
            pltpu.async_copy(xpad_hbm.at[srcv[p].at[pl.ds(j * S, S)]],
                             rows[p].at[pl.ds(j * S, S)], gsem)

    def drain_gathers(p):
        for j in range(J):
            pltpu.make_async_copy(xpad_hbm.at[srcv[p].at[pl.ds(j * S, S)]],
                                  rows[p].at[pl.ds(j * S, S)], gsem).wait()

    def fire_scatters(p):
        for j in range(J):
            pltpu.async_copy(rows[p].at[pl.ds(j * S, S)],
                             acc.at[dstv[p].at[j]], ssem, add=True)

    def drain_scatters(p):
        for j in range(J):
            pltpu.make_async_copy(rows[p].at[pl.ds(j * S, S)],
                                  acc.at[dstv[p].at[j]], ssem).wait()

    def fire_idx(cid, p):
        e0 = cid * CHUNK
        pltpu.async_copy(ei_hbm.at[0, pl.ds(e0, CHUNK)], srcv[p], isem)
        for j in range(J):
            pltpu.async_copy(ei_hbm.at[1, pl.ds(e0 + j * S, S)],
                             dstv[p].at[j], isem)

    def drain_idx(cid, p):
        e0 = cid * CHUNK
        pltpu.make_async_copy(ei_hbm.at[0, pl.ds(e0, CHUNK)], srcv[p],
                              isem).wait()
        for j in range(J):
            pltpu.make_async_copy(ei_hbm.at[1, pl.ds(e0 + j * S, S)],
                                  dstv[p].at[j], isem).wait()

    fire_idx(wid, 0)
    drain_idx(wid, 0)
    fire_gathers(0)

    @pl.loop(0, KFULL // 2)
    def _pair(t):
        a = (2 * t) * NW + wid
        b = a + NW

        @pl.when(t >= 1)
        def _():
            drain_scatters(1)
        fire_idx(b, 1)
        drain_gathers(0)
        fire_scatters(0)
        drain_idx(b, 1)
        fire_gathers(1)

        drain_scatters(0)

        @pl.when(t < KFULL // 2 - 1)
        def _():
            fire_idx(b + NW, 0)
        drain_gathers(1)
        fire_scatters(1)

        @pl.when(t < KFULL // 2 - 1)
        def _():
            drain_idx(b + NW, 0)
            fire_gathers(0)

    drain_scatters(1)

    @pl.when(wid < NREM)
    def _():
        fire_idx(KFULL * NW + wid, 0)
        drain_idx(KFULL * NW + wid, 0)
        fire_gathers(0)
        drain_gathers(0)
        fire_scatters(0)
        drain_scatters(0)

    plsc.subcore_barrier()

    @pl.loop(0, NZROUNDS)
    def _drain(k):
        m = k * NS + s
        off = m * ZCHUNK

        @pl.when(off + ZCHUNK <= N_NODES)
        def _():
            pltpu.sync_copy(acc.at[pl.ds(off, ZCHUNK)],
                            out_hbm.at[pl.ds(c * N_NODES + off, ZCHUNK)])

        @pl.when(m == N_FULL_Z)
        def _():
            pltpu.sync_copy(
                acc.at[pl.ds(N_FULL_Z * ZCHUNK, Z_TAIL)],
                out_hbm.at[pl.ds(c * N_NODES + N_FULL_Z * ZCHUNK, Z_TAIL)])


PROWS = N_NODES * DP // 128
FB_ROWS = 512
FGRID = -(-PROWS // FB_ROWS)


def _finish_body(p_ref, psel_ref, mbd_ref, o_ref):
    sums = p_ref[0] + p_ref[1]
    cnt = jnp.dot(sums, psel_ref[...],
                  preferred_element_type=jnp.float32)
    agg = sums / cnt
    out = jnp.dot(agg, mbd_ref[...], preferred_element_type=jnp.float32)
    o_ref[...] = jnp.maximum(out, 0.0)


def kernel(x, edge_index, W, B):
    x16 = jnp.concatenate(
        [x, jnp.ones((N_NODES, 1), jnp.float32),
         jnp.zeros((N_NODES, DP - D_IN - 1), jnp.float32)], axis=1)

    zeros2 = jnp.zeros((ZCHUNK, DP), jnp.float32)
    partials = _edge_aggregate(x16, edge_index, zeros2)
    p128 = partials.reshape(2, PROWS, 128)

    mp = jnp.zeros((DP, DP), jnp.float32)
    mp = mp.at[:D_IN, :D_IN].set((W + B).T)
    mbd = jnp.kron(jnp.eye(8, dtype=jnp.float32), mp)
    psel = jnp.asarray(_PSEL)

    out128 = pl.pallas_call(
        _finish_body,
        grid=(FGRID,),
        in_specs=[
            pl.BlockSpec((2, FB_ROWS, 128), lambda g: (0, g, 0)),
            pl.BlockSpec((128, 128), lambda g: (0, 0)),
            pl.BlockSpec((128, 128), lambda g: (0, 0)),
        ],
        out_specs=pl.BlockSpec((FB_ROWS, 128), lambda g: (g, 0)),
        out_shape=jax.ShapeDtypeStruct((PROWS, 128), jnp.float32),
    )(p128, psel, mbd)
    return out128.reshape(N_NODES, DP)[:, :D_IN]


# speedup vs baseline: 83.7392x; 1.0311x over previous
import functools

import jax
import jax.numpy as jnp
import numpy as np
from jax import lax
from jax.experimental import pallas as pl
from jax.experimental.pallas import tpu as pltpu
from jax.experimental.pallas import tpu_sc as plsc

N_NODES = 100000
N_EDGES = 6400000
D_IN = 10
DP = 16

NC = 2
NS = 16
NW = NC * NS

S = 128
J = 5
CHUNK = J * S
NCHUNKS = N_EDGES // CHUNK
KFULL = 312
NREM = NCHUNKS - KFULL * NW

ZCHUNK = 1280
NZROUNDS = -(-N_NODES // (NS * ZCHUNK))
N_FULL_Z = N_NODES // ZCHUNK
Z_TAIL = N_NODES - N_FULL_Z * ZCHUNK

_MESH = plsc.VectorSubcoreMesh(core_axis_name="c", subcore_axis_name="s")

_PSEL = np.zeros((128, 128), np.float32)
for _i in range(128):
    _PSEL[16 * (_i // 16) + D_IN, _i] = 1.0


@functools.partial(
    pl.kernel,
    out_type=jax.ShapeDtypeStruct((NC * N_NODES, DP), jnp.float32),
    mesh=_MESH,
    scratch_types=[
        pltpu.VMEM((J, S), jnp.int32),
        pltpu.VMEM((J, S), jnp.int32),
        pltpu.VMEM((J, S), jnp.int32),
        pltpu.VMEM((J, S), jnp.int32),
        pltpu.VMEM((CHUNK, DP), jnp.float32),
        pltpu.VMEM((CHUNK, DP), jnp.float32),
        pltpu.VMEM_SHARED((N_NODES, DP), jnp.float32),
        pltpu.SemaphoreType.DMA,
        pltpu.SemaphoreType.DMA,
        pltpu.SemaphoreType.DMA,
    ],
    compiler_params=pltpu.CompilerParams(use_tc_tiling_on_sc=False),
)
def _edge_aggregate(xpad_hbm, ei_hbm, zeros_hbm, out_hbm,
                    srcv0, srcv1, dstv0, dstv1, rows0, rows1,
                    acc, gsem, ssem, isem):
    c = lax.axis_index("c")
    s = lax.axis_index("s")
    wid = c * NS + s

    srcv = (srcv0, srcv1)
    dstv = (dstv0, dstv1)
    rows = (rows0, rows1)

    @pl.loop(0, NZROUNDS)
    def _zero(k):
        m = k * NS + s
        off = m * ZCHUNK

        @pl.when(off + ZCHUNK <= N_NODES)
        def _():
            pltpu.sync_copy(zeros_hbm, acc.at[pl.ds(off, ZCHUNK)])

        @pl.when(m == N_FULL_Z)
        def _():
            pltpu.sync_copy(zeros_hbm.at[pl.ds(0, Z_TAIL)],
                            acc.at[pl.ds(N_FULL_Z * ZCHUNK, Z_TAIL)])

    plsc.subcore_barrier()

    def fire_gathers(p):
        for j in range(J):
            pltpu.async_copy(xpad_hbm.at[srcv[p].at[j]],
                             rows[p].at[pl.ds(j * S, S)], gsem)

    def drain_gathers(p):
        for j in range(J):
            pltpu.make_async_copy(xpad_hbm.at[srcv[p].at[j]],
                                  rows[p].at[pl.ds(j * S, S)], gsem).wait()

    def fire_scatters(p):
        for j in range(J):
            pltpu.async_copy(rows[p].at[pl.ds(j * S, S)],
                             acc.at[dstv[p].at[j]], ssem, add=True)

    def drain_scatters(p):
        for j in range(J):
            pltpu.make_async_copy(rows[p].at[pl.ds(j * S, S)],
                                  acc.at[dstv[p].at[j]], ssem).wait()

    def fire_idx(cid, p):
        r0 = cid * J
        for j in range(J):
            pltpu.async_copy(ei_hbm.at[r0 + j, 0], srcv[p].at[j], isem)
            pltpu.async_copy(ei_hbm.at[r0 + j, 1], dstv[p].at[j], isem)

    def drain_idx(cid, p):
        r0 = cid * J
        for j in range(J):
            pltpu.make_async_copy(ei_hbm.at[r0 + j, 0], srcv[p].at[j],
                                  isem).wait()
            pltpu.make_async_copy(ei_hbm.at[r0 + j, 1], dstv[p].at[j],
                                  isem).wait()

    fire_idx(wid, 0)
    drain_idx(wid, 0)
    fire_gathers(0)

    @pl.loop(0, KFULL // 2)
    def _pair(t):
        a = (2 * t) * NW + wid
        b = a + NW

        @pl.when(t >= 1)
        def _():
            drain_scatters(1)
        fire_idx(b, 1)
        drain_gathers(0)
        fire_scatters(0)
        drain_idx(b, 1)
        fire_gathers(1)

        drain_scatters(0)

        @pl.when(t < KFULL // 2 - 1)
        def _():
            fire_idx(b + NW, 0)
        drain_gathers(1)
        fire_scatters(1)

        @pl.when(t < KFULL // 2 - 1)
        def _():
            drain_idx(b + NW, 0)
            fire_gathers(0)

    drain_scatters(1)

    @pl.when(wid < NREM)
    def _():
        fire_idx(KFULL * NW + wid, 0)
        drain_idx(KFULL * NW + wid, 0)
        fire_gathers(0)
        drain_gathers(0)
        fire_scatters(0)
        drain_scatters(0)

    plsc.subcore_barrier()

    @pl.loop(0, NZROUNDS)
    def _drain(k):
        m = k * NS + s
        off = m * ZCHUNK

        @pl.when(off + ZCHUNK <= N_NODES)
        def _():
            pltpu.sync_copy(acc.at[pl.ds(off, ZCHUNK)],
                            out_hbm.at[pl.ds(c * N_NODES + off, ZCHUNK)])

        @pl.when(m == N_FULL_Z)
        def _():
            pltpu.sync_copy(
                acc.at[pl.ds(N_FULL_Z * ZCHUNK, Z_TAIL)],
                out_hbm.at[pl.ds(c * N_NODES + N_FULL_Z * ZCHUNK, Z_TAIL)])


PROWS = N_NODES * DP // 128
FB_ROWS = 512
FGRID = -(-PROWS // FB_ROWS)


def _finish_body(p_ref, psel_ref, mbd_ref, o_ref):
    sums = p_ref[0] + p_ref[1]
    cnt = jnp.dot(sums, psel_ref[...],
                  preferred_element_type=jnp.float32)
    agg = sums / cnt
    out = jnp.dot(agg, mbd_ref[...], preferred_element_type=jnp.float32)
    o_ref[...] = jnp.maximum(out, 0.0)


def kernel(x, edge_index, W, B):
    x16 = jnp.concatenate(
        [x, jnp.ones((N_NODES, 1), jnp.float32),
         jnp.zeros((N_NODES, DP - D_IN - 1), jnp.float32)], axis=1)

    zeros2 = jnp.zeros((ZCHUNK, DP), jnp.float32)
    ei3 = edge_index.reshape(2, N_EDGES // S, S).transpose(1, 0, 2)
    partials = _edge_aggregate(x16, ei3, zeros2)
    p128 = partials.reshape(2, PROWS, 128)

    mp = jnp.zeros((DP, DP), jnp.float32)
    mp = mp.at[:D_IN, :D_IN].set((W + B).T)
    mbd = jnp.kron(jnp.eye(8, dtype=jnp.float32), mp)
    psel = jnp.asarray(_PSEL)

    out128 = pl.pallas_call(
        _finish_body,
        grid=(FGRID,),
        in_specs=[
            pl.BlockSpec((2, FB_ROWS, 128), lambda g: (0, g, 0)),
            pl.BlockSpec((128, 128), lambda g: (0, 0)),
            pl.BlockSpec((128, 128), lambda g: (0, 0)),
        ],
        out_specs=pl.BlockSpec((FB_ROWS, 128), lambda g: (g, 0)),
        out_shape=jax.ShapeDtypeStruct((PROWS, 128), jnp.float32),
    )(p128, psel, mbd)
    return out128.reshape(N_NODES, DP)[:, :D_IN]


# speedup vs baseline: 84.2528x; 1.0061x over previous
import functools

import jax
import jax.numpy as jnp
import numpy as np
from jax import lax
from jax.experimental import pallas as pl
from jax.experimental.pallas import tpu as pltpu
from jax.experimental.pallas import tpu_sc as plsc

N_NODES = 100000
N_EDGES = 6400000
D_IN = 10
DP = 16

NC = 2
NS = 16
NW = NC * NS

S = 128
J = 5
CHUNK = J * S
NCHUNKS = N_EDGES // CHUNK
KFULL = 312
NREM = NCHUNKS - KFULL * NW

ZCHUNK = 1280
NZROUNDS = -(-N_NODES // (NS * ZCHUNK))
N_FULL_Z = N_NODES // ZCHUNK
Z_TAIL = N_NODES - N_FULL_Z * ZCHUNK

_MESH = plsc.VectorSubcoreMesh(core_axis_name="c", subcore_axis_name="s")

_PSEL = np.zeros((128, 128), np.float32)
for _i in range(128):
    _PSEL[16 * (_i // 16) + D_IN, _i] = 1.0


@functools.partial(
    pl.kernel,
    out_type=jax.ShapeDtypeStruct((NC * N_NODES, DP), jnp.float32),
    mesh=_MESH,
    scratch_types=[
        pltpu.VMEM((J, 2, S), jnp.int32),
        pltpu.VMEM((J, 2, S), jnp.int32),
        pltpu.VMEM((CHUNK, DP), jnp.float32),
        pltpu.VMEM((CHUNK, DP), jnp.float32),
        pltpu.VMEM_SHARED((N_NODES, DP), jnp.float32),
        pltpu.SemaphoreType.DMA,
        pltpu.SemaphoreType.DMA,
        pltpu.SemaphoreType.DMA,
    ],
    compiler_params=pltpu.CompilerParams(use_tc_tiling_on_sc=False),
)
def _edge_aggregate(xpad_hbm, ei_hbm, zeros_hbm, out_hbm,
                    sdv0, sdv1, rows0, rows1,
                    acc, gsem, ssem, isem):
    c = lax.axis_index("c")
    s = lax.axis_index("s")
    wid = c * NS + s

    sdv = (sdv0, sdv1)
    rows = (rows0, rows1)

    @pl.loop(0, NZROUNDS)
    def _zero(k):
        m = k * NS + s
        off = m * ZCHUNK

        @pl.when(off + ZCHUNK <= N_NODES)
        def _():
            pltpu.sync_copy(zeros_hbm, acc.at[pl.ds(off, ZCHUNK)])

        @pl.when(m == N_FULL_Z)
        def _():
            pltpu.sync_copy(zeros_hbm.at[pl.ds(0, Z_TAIL)],
                            acc.at[pl.ds(N_FULL_Z * ZCHUNK, Z_TAIL)])

    plsc.subcore_barrier()

    def fire_gathers(p):
        for j in range(J):
            pltpu.async_copy(xpad_hbm.at[sdv[p].at[j, 0]],
                             rows[p].at[pl.ds(j * S, S)], gsem)

    def drain_gathers(p):
        for j in range(J):
            pltpu.make_async_copy(xpad_hbm.at[sdv[p].at[j, 0]],
                                  rows[p].at[pl.ds(j * S, S)], gsem).wait()

    def fire_scatters(p):
        for j in range(J):
            pltpu.async_copy(rows[p].at[pl.ds(j * S, S)],
                             acc.at[sdv[p].at[j, 1]], ssem, add=True)

    def drain_scatters(p):
        for j in range(J):
            pltpu.make_async_copy(rows[p].at[pl.ds(j * S, S)],
                                  acc.at[sdv[p].at[j, 1]], ssem).wait()

    def fire_idx(cid, p):
        r0 = cid * J
        for j in range(J):
            pltpu.async_copy(ei_hbm.at[r0 + j], sdv[p].at[j], isem)

    def drain_idx(cid, p):
        r0 = cid * J
        for j in range(J):
            pltpu.make_async_copy(ei_hbm.at[r0 + j], sdv[p].at[j],
                                  isem).wait()

    fire_idx(wid, 0)
    drain_idx(wid, 0)
    fire_gathers(0)

    @pl.loop(0, KFULL // 2)
    def _pair(t):
        a = (2 * t) * NW + wid
        b = a + NW

        @pl.when(t >= 1)
        def _():
            drain_scatters(1)
        fire_idx(b, 1)
        drain_gathers(0)
        fire_scatters(0)
        drain_idx(b, 1)
        fire_gathers(1)

        drain_scatters(0)

        @pl.when(t < KFULL // 2 - 1)
        def _():
            fire_idx(b + NW, 0)
        drain_gathers(1)
        fire_scatters(1)

        @pl.when(t < KFULL // 2 - 1)
        def _():
            drain_idx(b + NW, 0)
            fire_gathers(0)

    drain_scatters(1)

    @pl.when(wid < NREM)
    def _():
        fire_idx(KFULL * NW + wid, 0)
        drain_idx(KFULL * NW + wid, 0)
        fire_gathers(0)
        drain_gathers(0)
        fire_scatters(0)
        drain_scatters(0)

    plsc.subcore_barrier()

    @pl.loop(0, NZROUNDS)
    def _drain(k):
        m = k * NS + s
        off = m * ZCHUNK

        @pl.when(off + ZCHUNK <= N_NODES)
        def _():
            pltpu.sync_copy(acc.at[pl.ds(off, ZCHUNK)],
                            out_hbm.at[pl.ds(c * N_NODES + off, ZCHUNK)])

        @pl.when(m == N_FULL_Z)
        def _():
            pltpu.sync_copy(
                acc.at[pl.ds(N_FULL_Z * ZCHUNK, Z_TAIL)],
                out_hbm.at[pl.ds(c * N_NODES + N_FULL_Z * ZCHUNK, Z_TAIL)])


PROWS = N_NODES * DP // 128
FB_ROWS = 512
FGRID = -(-PROWS // FB_ROWS)


def _finish_body(p_ref, psel_ref, mbd_ref, o_ref):
    sums = p_ref[0] + p_ref[1]
    cnt = jnp.dot(sums, psel_ref[...],
                  preferred_element_type=jnp.float32)
    agg = sums / cnt
    out = jnp.dot(agg, mbd_ref[...], preferred_element_type=jnp.float32)
    o_ref[...] = jnp.maximum(out, 0.0)


def kernel(x, edge_index, W, B):
    x16 = jnp.concatenate(
        [x, jnp.ones((N_NODES, 1), jnp.float32),
         jnp.zeros((N_NODES, DP - D_IN - 1), jnp.float32)], axis=1)

    zeros2 = jnp.zeros((ZCHUNK, DP), jnp.float32)
    ei3 = edge_index.reshape(2, N_EDGES // S, S).transpose(1, 0, 2)
    partials = _edge_aggregate(x16, ei3, zeros2)
    p128 = partials.reshape(2, PROWS, 128)

    mp = jnp.zeros((DP, DP), jnp.float32)
    mp = mp.at[:D_IN, :D_IN].set((W + B).T)
    mbd = jnp.kron(jnp.eye(8, dtype=jnp.float32), mp)
    psel = jnp.asarray(_PSEL)

    out128 = pl.pallas_call(
        _finish_body,
        grid=(FGRID,),
        in_specs=[
            pl.BlockSpec((2, FB_ROWS, 128), lambda g: (0, g, 0)),
            pl.BlockSpec((128, 128), lambda g: (0, 0)),
            pl.BlockSpec((128, 128), lambda g: (0, 0)),
        ],
        out_specs=pl.BlockSpec((FB_ROWS, 128), lambda g: (g, 0)),
        out_shape=jax.ShapeDtypeStruct((PROWS, 128), jnp.float32),
    )(p128, psel, mbd)
    return out128.reshape(N_NODES, DP)[:, :D_IN]
